# baseline (speedup 1.0000x reference)
import functools

import jax
import jax.numpy as jnp
from jax.experimental import pallas as pl
from jax.experimental.pallas import tpu as pltpu
from jax.experimental.pallas import tpu_sc as plsc

NB = 16
E_TILE = 4096
GATHER_W = 256
EDGE_T = 2000


def _pw_mlp_body(pf_ref, w0_ref, b0_ref, w1_ref, b1_ref, w2_ref, b2_ref,
                 out_ref):
    p = pf_ref[...]
    h = jnp.maximum(
        jnp.dot(p, w0_ref[...], preferred_element_type=jnp.float32)
        + b0_ref[...], 0.0)
    h = jnp.maximum(
        jnp.dot(h, w1_ref[...], preferred_element_type=jnp.float32)
        + b1_ref[...], 0.0)
    h = jnp.maximum(
        jnp.dot(h, w2_ref[...], preferred_element_type=jnp.float32)
        + b2_ref[...], 0.0)
    out_ref[...] = h


def _pairwise_mlp(pairFeatRaw, pw_W0, pw_b0, pw_W1, pw_b1, pw_W2, pw_b2):
    E = pairFeatRaw.shape[0]
    grid = (E // E_TILE,)
    full = lambda *s: pl.BlockSpec(s, lambda i: tuple(0 for _ in s))
    return pl.pallas_call(
        _pw_mlp_body,
        grid=grid,
        in_specs=[
            pl.BlockSpec((E_TILE, 9), lambda i: (i, 0)),
            full(9, 256), full(256), full(256, 256), full(256),
            full(256, 32), full(32),
        ],
        out_specs=pl.BlockSpec((E_TILE, 32), lambda i: (i, 0)),
        out_shape=jax.ShapeDtypeStruct((E, 32), jnp.float32),
    )(pairFeatRaw, pw_W0, pw_b0, pw_W1, pw_b1, pw_W2, pw_b2)


def _node_table_body(x_ref, w1_ref, b1_ref, wc_ref, wn_ref, t_ref):
    f1 = jnp.maximum(
        jnp.dot(x_ref[...], w1_ref[...], preferred_element_type=jnp.float32)
        + b1_ref[...], 0.0)
    cg = jnp.dot(f1, wc_ref[...], preferred_element_type=jnp.float32)
    nh = jnp.dot(f1, wn_ref[...], preferred_element_type=jnp.float32)
    t_ref[...] = jnp.concatenate([cg, nh], axis=1)


def _node_table(x, w1, b1, wc, wn):
    N = x.shape[0]
    full = lambda *s: pl.BlockSpec(s, lambda: tuple(0 for _ in s))
    return pl.pallas_call(
        _node_table_body,
        in_specs=[full(N, 128), full(128, 32), full(1, 32), full(32, 64),
                  full(32, 64)],
        out_specs=full(N, 128),
        out_shape=jax.ShapeDtypeStruct((N, 128), jnp.float32),
    )(x, w1, b1, wc, wn)


def _update_and_table_body(x_ref, pooled_ref, wo1_ref, bo1_ref, wo2_ref,
                           bo2_ref, wout_ref, bout_ref, w1_ref, b1_ref,
                           wc_ref, wn_ref, x_out_ref, t_ref):
    po = jnp.maximum(
        jnp.dot(pooled_ref[:x_ref.shape[0], :64], wo1_ref[...],
                preferred_element_type=jnp.float32) + bo1_ref[...], 0.0)
    po = jnp.maximum(
        jnp.dot(po, wo2_ref[...], preferred_element_type=jnp.float32)
        + bo2_ref[...], 0.0)
    refined = jnp.dot(po, wout_ref[...],
                      preferred_element_type=jnp.float32) + bout_ref[...]
    x_new = jnp.maximum(x_ref[...] + refined, 0.0)
    x_out_ref[...] = x_new
    f1 = jnp.maximum(
        jnp.dot(x_new, w1_ref[...], preferred_element_type=jnp.float32)
        + b1_ref[...], 0.0)
    cg = jnp.dot(f1, wc_ref[...], preferred_element_type=jnp.float32)
    nh = jnp.dot(f1, wn_ref[...], preferred_element_type=jnp.float32)
    t_ref[...] = jnp.concatenate([cg, nh], axis=1)


def _update_and_table(x, pooled, wo1, bo1, wo2, bo2, wout, bout, w1, b1,
                      wc, wn):
    N = x.shape[0]
    full = lambda *s: pl.BlockSpec(s, lambda: tuple(0 for _ in s))
    return pl.pallas_call(
        _update_and_table_body,
        in_specs=[full(N, 128), full(pooled.shape[0], 128), full(64, 64),
                  full(1, 64),
                  full(64, 64), full(1, 64), full(64, 128), full(1, 128),
                  full(128, 32), full(1, 32), full(32, 64), full(32, 64)],
        out_specs=[full(N, 128), full(N, 128)],
        out_shape=[jax.ShapeDtypeStruct((N, 128), jnp.float32),
                   jax.ShapeDtypeStruct((N, 128), jnp.float32)],
    )(x, pooled, wo1, bo1, wo2, bo2, wout, bout, w1, b1, wc, wn)


def _final_update_body(x_ref, pooled_ref, wo1_ref, bo1_ref, wo2_ref,
                       bo2_ref, wout_ref, bout_ref, x_out_ref):
    po = jnp.maximum(
        jnp.dot(pooled_ref[:x_ref.shape[0], :64], wo1_ref[...],
                preferred_element_type=jnp.float32) + bo1_ref[...], 0.0)
    po = jnp.maximum(
        jnp.dot(po, wo2_ref[...], preferred_element_type=jnp.float32)
        + bo2_ref[...], 0.0)
    refined = jnp.dot(po, wout_ref[...],
                      preferred_element_type=jnp.float32) + bout_ref[...]
    x_out_ref[...] = jnp.maximum(x_ref[...] + refined, 0.0)


def _final_update(x, pooled, wo1, bo1, wo2, bo2, wout, bout):
    N = x.shape[0]
    full = lambda *s: pl.BlockSpec(s, lambda: tuple(0 for _ in s))
    return pl.pallas_call(
        _final_update_body,
        in_specs=[full(N, 128), full(pooled.shape[0], 128), full(64, 64),
                  full(1, 64),
                  full(64, 64), full(1, 64), full(64, 128), full(1, 128)],
        out_specs=full(N, 128),
        out_shape=jax.ShapeDtypeStruct((N, 128), jnp.float32),
    )(x, pooled, wo1, bo1, wo2, bo2, wout, bout)


def _sc_gather(table, idx2d):
    M = idx2d.shape[1]
    mesh = plsc.VectorSubcoreMesh(core_axis_name="core",
                                  subcore_axis_name="subcore")

    @functools.partial(
        pl.kernel,
        out_type=jax.ShapeDtypeStruct((M, 128), jnp.float32),
        mesh=mesh)
    def k(tab_hbm, i_hbm, o_hbm):
        def body(i_v, o_v):
            pltpu.sync_copy(tab_hbm.at[i_v.at[0]], o_v)

        pltpu.emit_pipeline(
            body,
            grid=(M // GATHER_W,),
            in_specs=[pl.BlockSpec((1, GATHER_W), lambda i: (0, i))],
            out_specs=[pl.BlockSpec((GATHER_W, 128), lambda i: (i, 0))],
            core_axis_name=("core", "subcore"),
            dimension_semantics=(pltpu.PARALLEL,),
        )(i_hbm, o_hbm)

    return k(table, idx2d)


def _edge_mlp_body(zc_ref, zn_ref, p_ref, wp_ref, b1_ref, w2_ref, b2_ref,
                   out_ref):
    u = zc_ref[:, :64] + zn_ref[:, 64:]
    pw = jnp.dot(p_ref[...], wp_ref[...], preferred_element_type=jnp.float32)
    c1 = jnp.maximum(u + pw + b1_ref[...], 0.0)
    c2 = jnp.maximum(
        jnp.dot(c1, w2_ref[...], preferred_element_type=jnp.float32)
        + b2_ref[...], 0.0)
    out_ref[...] = jnp.concatenate([c2, c2], axis=1)


def _edge_mlp(z, p, wp, b1, w2, b2, E):
    nblk = E // EDGE_T
    full = lambda *s: pl.BlockSpec(s, lambda i: tuple(0 for _ in s))
    return pl.pallas_call(
        _edge_mlp_body,
        grid=(nblk,),
        in_specs=[
            pl.BlockSpec((EDGE_T, 128), lambda i: (i, 0)),
            pl.BlockSpec((EDGE_T, 128), lambda i: (i + nblk, 0)),
            pl.BlockSpec((EDGE_T, 32), lambda i: (i, 0)),
            full(32, 64), full(1, 64), full(64, 64), full(1, 64),
        ],
        out_specs=pl.BlockSpec((EDGE_T, 128), lambda i: (i, 0)),
        out_shape=jax.ShapeDtypeStruct((E + SEG_CH, 128), jnp.float32),
    )(z, z, p, wp, b1, w2, b2)


NPW = 320
SEG_CH = 512


def _sc_segmax(comb2, cIdxs, rs_pad, N, E):
    nout = 32 * NPW
    mesh = plsc.VectorSubcoreMesh(core_axis_name="core",
                                  subcore_axis_name="subcore")

    @functools.partial(
        pl.kernel,
        out_type=jax.ShapeDtypeStruct((nout, 128), jnp.float32),
        mesh=mesh,
        scratch_types=[
            pltpu.VMEM((SEG_CH, 128), jnp.float32),
            pltpu.VMEM((NPW + 8, 128), jnp.float32),
            pltpu.VMEM((SEG_CH,), jnp.int32),
            pltpu.VMEM((NPW + 24,), jnp.int32),
        ])
    def k(c2_hbm, ci_hbm, rs_hbm, o_hbm, chunk_v, pool_v, c_v, rs_v):
        wid = jax.lax.axis_index("subcore") * 2 + jax.lax.axis_index("core")
        nlo = wid * NPW
        nnodes = jnp.minimum(NPW, jnp.maximum(0, N - nlo))
        nhi = nlo + nnodes
        pltpu.sync_copy(rs_hbm.at[pl.ds(nlo, NPW + 8)],
                        rs_v.at[pl.ds(0, NPW + 8)])

        zv = jnp.zeros((16,), jnp.float32)

        @pl.loop(0, NPW)
        def _(r):
            @pl.loop(0, 128, step=16)
            def _(kk):
                pool_v[r, pl.ds(kk, 16)] = zv

        e_begin = rs_v[pl.ds(0, 16)][0]
        e_end = rs_v[pl.ds(nnodes, 16)][0]
        abegin = (e_begin // 16) * 16
        nch = (e_end - abegin + SEG_CH - 1) // SEG_CH

        def chunk_body(ch, st):
            base = abegin + ch * SEG_CH
            pltpu.sync_copy(c2_hbm.at[pl.ds(base, SEG_CH)], chunk_v)
            pltpu.sync_copy(ci_hbm.at[pl.ds(base, SEG_CH)], c_v)

            def group_body(g, st2):
                cvec = c_v[pl.ds(g * 16, 16)]
                st3 = st2
                for l in range(16):
                    cprev, a0, a1, a2, a3 = st3
                    ce = cvec[l]
                    same = ce == cprev
                    row = g * 16 + l
                    v0 = chunk_v[row, pl.ds(0, 16)]
                    v1 = chunk_v[row, pl.ds(16, 16)]
                    v2 = chunk_v[row, pl.ds(32, 16)]
                    v3 = chunk_v[row, pl.ds(48, 16)]
                    a0 = jnp.where(same, jnp.maximum(a0, v0), v0)
                    a1 = jnp.where(same, jnp.maximum(a1, v1), v1)
                    a2 = jnp.where(same, jnp.maximum(a2, v2), v2)
                    a3 = jnp.where(same, jnp.maximum(a3, v3), v3)
                    inr = jnp.logical_and(ce >= nlo, ce < nhi)
                    lrow = jnp.where(inr, ce - nlo, NPW)
                    pool_v[lrow, pl.ds(0, 16)] = a0
                    pool_v[lrow, pl.ds(16, 16)] = a1
                    pool_v[lrow, pl.ds(32, 16)] = a2
                    pool_v[lrow, pl.ds(48, 16)] = a3
                    st3 = (ce, a0, a1, a2, a3)
                return st3

            return jax.lax.fori_loop(0, SEG_CH // 16, group_body, st)

        st0 = (jnp.int32(-1), zv, zv, zv, zv)
        jax.lax.fori_loop(0, nch, chunk_body, st0)
        pltpu.sync_copy(pool_v.at[pl.ds(0, NPW)], o_hbm.at[pl.ds(nlo, NPW)])

    return k(comb2, cIdxs, rs_pad)


def kernel(detFeatures, cIdxs, nIdxs, pairFeatRaw, pw_W0, pw_b0, pw_W1,
           pw_b1, pw_W2, pw_b2, blk_fc1_W, blk_fc1_b, blk_pw1_W, blk_pw1_b,
           blk_pw2_W, blk_pw2_b, blk_po1_W, blk_po1_b, blk_po2_W, blk_po2_b,
           blk_out_W, blk_out_b):
    E = cIdxs.shape[0]
    N = detFeatures.shape[0]
    p = _pairwise_mlp(pairFeatRaw, pw_W0, pw_b0, pw_W1, pw_b1, pw_W2, pw_b2)
    allIdx = jnp.concatenate([cIdxs, nIdxs]).reshape(1, 2 * E)
    rs = jnp.searchsorted(cIdxs, jnp.arange(N + 1, dtype=jnp.int32)
                          ).astype(jnp.int32)
    rs_pad = jnp.concatenate(
        [rs, jnp.full((32 * NPW + 8 - (N + 1),), E, jnp.int32)])
    ci_pad = jnp.concatenate([cIdxs, jnp.full((SEG_CH,), N, jnp.int32)])

    x = detFeatures
    pooled = None
    for i in range(NB):
        w1 = blk_fc1_W[i]
        b1 = blk_fc1_b[i].reshape(1, 32)
        wp = blk_pw1_W[i, :32, :]
        wc = blk_pw1_W[i, 32:64, :]
        wn = blk_pw1_W[i, 64:, :]
        pb1 = blk_pw1_b[i].reshape(1, 64)
        w2 = blk_pw2_W[i]
        pb2 = blk_pw2_b[i].reshape(1, 64)
        if i == 0:
            table = _node_table(x, w1, b1, wc, wn)
        else:
            x, table = _update_and_table(
                x, pooled, blk_po1_W[i - 1], blk_po1_b[i - 1].reshape(1, 64),
                blk_po2_W[i - 1], blk_po2_b[i - 1].reshape(1, 64),
                blk_out_W[i - 1], blk_out_b[i - 1].reshape(1, 128),
                w1, b1, wc, wn)
        z = _sc_gather(table, allIdx)
        comb2 = _edge_mlp(z, p, wp, pb1, w2, pb2, E)
        pooled = _sc_segmax(comb2, ci_pad, rs_pad, N, E)
    x = _final_update(
        x, pooled, blk_po1_W[NB - 1], blk_po1_b[NB - 1].reshape(1, 64),
        blk_po2_W[NB - 1], blk_po2_b[NB - 1].reshape(1, 64),
        blk_out_W[NB - 1], blk_out_b[NB - 1].reshape(1, 128),
    )
    return x


# speedup vs baseline: 1.8722x; 1.8722x over previous
import functools

import jax
import jax.numpy as jnp
from jax.experimental import pallas as pl
from jax.experimental.pallas import tpu as pltpu
from jax.experimental.pallas import tpu_sc as plsc

NB = 16
E_TILE = 4096
GATHER_W = 256
EDGE_T = 2000


def _pw_mlp_body(pf_ref, w0_ref, b0_ref, w1_ref, b1_ref, w2_ref, b2_ref,
                 out_ref):
    p = pf_ref[...]
    h = jnp.maximum(
        jnp.dot(p, w0_ref[...], preferred_element_type=jnp.float32)
        + b0_ref[...], 0.0)
    h = jnp.maximum(
        jnp.dot(h, w1_ref[...], preferred_element_type=jnp.float32)
        + b1_ref[...], 0.0)
    h = jnp.maximum(
        jnp.dot(h, w2_ref[...], preferred_element_type=jnp.float32)
        + b2_ref[...], 0.0)
    out_ref[...] = h


def _pairwise_mlp(pairFeatRaw, pw_W0, pw_b0, pw_W1, pw_b1, pw_W2, pw_b2):
    E = pairFeatRaw.shape[0]
    grid = (E // E_TILE,)
    full = lambda *s: pl.BlockSpec(s, lambda i: tuple(0 for _ in s))
    return pl.pallas_call(
        _pw_mlp_body,
        grid=grid,
        in_specs=[
            pl.BlockSpec((E_TILE, 9), lambda i: (i, 0)),
            full(9, 256), full(256), full(256, 256), full(256),
            full(256, 32), full(32),
        ],
        out_specs=pl.BlockSpec((E_TILE, 32), lambda i: (i, 0)),
        out_shape=jax.ShapeDtypeStruct((E, 32), jnp.float32),
    )(pairFeatRaw, pw_W0, pw_b0, pw_W1, pw_b1, pw_W2, pw_b2)


def _node_table_body(x_ref, w1_ref, b1_ref, wc_ref, wn_ref, t_ref):
    f1 = jnp.maximum(
        jnp.dot(x_ref[...], w1_ref[...], preferred_element_type=jnp.float32)
        + b1_ref[...], 0.0)
    cg = jnp.dot(f1, wc_ref[...], preferred_element_type=jnp.float32)
    nh = jnp.dot(f1, wn_ref[...], preferred_element_type=jnp.float32)
    t_ref[...] = jnp.concatenate([cg, nh], axis=1)


def _node_table(x, w1, b1, wc, wn):
    N = x.shape[0]
    full = lambda *s: pl.BlockSpec(s, lambda: tuple(0 for _ in s))
    return pl.pallas_call(
        _node_table_body,
        in_specs=[full(N, 128), full(128, 32), full(1, 32), full(32, 64),
                  full(32, 64)],
        out_specs=full(N, 128),
        out_shape=jax.ShapeDtypeStruct((N, 128), jnp.float32),
    )(x, w1, b1, wc, wn)


def _update_and_table_body(x_ref, pooled_ref, wo1_ref, bo1_ref, wo2_ref,
                           bo2_ref, wout_ref, bout_ref, w1_ref, b1_ref,
                           wc_ref, wn_ref, x_out_ref, t_ref):
    po = jnp.maximum(
        jnp.dot(pooled_ref[:x_ref.shape[0], :64], wo1_ref[...],
                preferred_element_type=jnp.float32) + bo1_ref[...], 0.0)
    po = jnp.maximum(
        jnp.dot(po, wo2_ref[...], preferred_element_type=jnp.float32)
        + bo2_ref[...], 0.0)
    refined = jnp.dot(po, wout_ref[...],
                      preferred_element_type=jnp.float32) + bout_ref[...]
    x_new = jnp.maximum(x_ref[...] + refined, 0.0)
    x_out_ref[...] = x_new
    f1 = jnp.maximum(
        jnp.dot(x_new, w1_ref[...], preferred_element_type=jnp.float32)
        + b1_ref[...], 0.0)
    cg = jnp.dot(f1, wc_ref[...], preferred_element_type=jnp.float32)
    nh = jnp.dot(f1, wn_ref[...], preferred_element_type=jnp.float32)
    t_ref[...] = jnp.concatenate([cg, nh], axis=1)


def _update_and_table(x, pooled, wo1, bo1, wo2, bo2, wout, bout, w1, b1,
                      wc, wn):
    N = x.shape[0]
    full = lambda *s: pl.BlockSpec(s, lambda: tuple(0 for _ in s))
    return pl.pallas_call(
        _update_and_table_body,
        in_specs=[full(N, 128), full(pooled.shape[0], 128), full(64, 64),
                  full(1, 64),
                  full(64, 64), full(1, 64), full(64, 128), full(1, 128),
                  full(128, 32), full(1, 32), full(32, 64), full(32, 64)],
        out_specs=[full(N, 128), full(N, 128)],
        out_shape=[jax.ShapeDtypeStruct((N, 128), jnp.float32),
                   jax.ShapeDtypeStruct((N, 128), jnp.float32)],
    )(x, pooled, wo1, bo1, wo2, bo2, wout, bout, w1, b1, wc, wn)


def _final_update_body(x_ref, pooled_ref, wo1_ref, bo1_ref, wo2_ref,
                       bo2_ref, wout_ref, bout_ref, x_out_ref):
    po = jnp.maximum(
        jnp.dot(pooled_ref[:x_ref.shape[0], :64], wo1_ref[...],
                preferred_element_type=jnp.float32) + bo1_ref[...], 0.0)
    po = jnp.maximum(
        jnp.dot(po, wo2_ref[...], preferred_element_type=jnp.float32)
        + bo2_ref[...], 0.0)
    refined = jnp.dot(po, wout_ref[...],
                      preferred_element_type=jnp.float32) + bout_ref[...]
    x_out_ref[...] = jnp.maximum(x_ref[...] + refined, 0.0)


def _final_update(x, pooled, wo1, bo1, wo2, bo2, wout, bout):
    N = x.shape[0]
    full = lambda *s: pl.BlockSpec(s, lambda: tuple(0 for _ in s))
    return pl.pallas_call(
        _final_update_body,
        in_specs=[full(N, 128), full(pooled.shape[0], 128), full(64, 64),
                  full(1, 64),
                  full(64, 64), full(1, 64), full(64, 128), full(1, 128)],
        out_specs=full(N, 128),
        out_shape=jax.ShapeDtypeStruct((N, 128), jnp.float32),
    )(x, pooled, wo1, bo1, wo2, bo2, wout, bout)


def _sc_gather(table, idx2d):
    M = idx2d.shape[1]
    mesh = plsc.VectorSubcoreMesh(core_axis_name="core",
                                  subcore_axis_name="subcore")

    H = GATHER_W // 2

    @functools.partial(
        pl.kernel,
        out_type=jax.ShapeDtypeStruct((M, 128), jnp.float32),
        mesh=mesh,
        scratch_types=[pltpu.SemaphoreType.DMA, pltpu.SemaphoreType.DMA])
    def k(tab_hbm, i_hbm, o_hbm, sem1, sem2):
        def body(i_v, o_v):
            h1 = pltpu.async_copy(
                tab_hbm.at[i_v.at[0, pl.ds(0, H)]], o_v.at[pl.ds(0, H)],
                sem1)
            h2 = pltpu.async_copy(
                tab_hbm.at[i_v.at[0, pl.ds(H, H)]], o_v.at[pl.ds(H, H)],
                sem2)
            h1.wait()
            h2.wait()

        pltpu.emit_pipeline(
            body,
            grid=(M // GATHER_W,),
            in_specs=[pl.BlockSpec((1, GATHER_W), lambda i: (0, i))],
            out_specs=[pl.BlockSpec((GATHER_W, 128), lambda i: (i, 0))],
            core_axis_name=("core", "subcore"),
            dimension_semantics=(pltpu.PARALLEL,),
        )(i_hbm, o_hbm)

    return k(table, idx2d)


def _edge_mlp_body(zc_ref, zn_ref, p_ref, wp_ref, b1_ref, w2_ref, b2_ref,
                   out_ref):
    u = zc_ref[:, :64] + zn_ref[:, 64:]
    pw = jnp.dot(p_ref[...], wp_ref[...], preferred_element_type=jnp.float32)
    c1 = jnp.maximum(u + pw + b1_ref[...], 0.0)
    c2 = jnp.maximum(
        jnp.dot(c1, w2_ref[...], preferred_element_type=jnp.float32)
        + b2_ref[...], 0.0)
    out_ref[...] = jnp.concatenate([c2, c2], axis=1)


def _edge_mlp(z, p, wp, b1, w2, b2, E):
    nblk = E // EDGE_T
    full = lambda *s: pl.BlockSpec(s, lambda i: tuple(0 for _ in s))
    return pl.pallas_call(
        _edge_mlp_body,
        grid=(nblk,),
        in_specs=[
            pl.BlockSpec((EDGE_T, 128), lambda i: (i, 0)),
            pl.BlockSpec((EDGE_T, 128), lambda i: (i + nblk, 0)),
            pl.BlockSpec((EDGE_T, 32), lambda i: (i, 0)),
            full(32, 64), full(1, 64), full(64, 64), full(1, 64),
        ],
        out_specs=pl.BlockSpec((EDGE_T, 128), lambda i: (i, 0)),
        out_shape=jax.ShapeDtypeStruct((E + SEG_CH, 128), jnp.float32),
    )(z, z, p, wp, b1, w2, b2)


NPW = 320
SEG_CH = 512


def _sc_segmax(comb2, cIdxs, rs_pad, N, E):
    nout = 32 * NPW
    mesh = plsc.VectorSubcoreMesh(core_axis_name="core",
                                  subcore_axis_name="subcore")

    @functools.partial(
        pl.kernel,
        out_type=jax.ShapeDtypeStruct((nout, 128), jnp.float32),
        mesh=mesh,
        scratch_types=[
            pltpu.VMEM((SEG_CH, 128), jnp.float32),
            pltpu.VMEM((NPW + 8, 128), jnp.float32),
            pltpu.VMEM((SEG_CH + 16,), jnp.int32),
            pltpu.VMEM((NPW + 24,), jnp.int32),
        ])
    def k(c2_hbm, ci_hbm, rs_hbm, o_hbm, chunk_v, pool_v, c_v, rs_v):
        wid = jax.lax.axis_index("subcore") * 2 + jax.lax.axis_index("core")
        nlo = wid * NPW
        nnodes = jnp.minimum(NPW, jnp.maximum(0, N - nlo))
        nhi = nlo + nnodes
        pltpu.sync_copy(rs_hbm.at[pl.ds(nlo, NPW + 8)],
                        rs_v.at[pl.ds(0, NPW + 8)])

        zv = jnp.zeros((16,), jnp.float32)

        @pl.loop(0, NPW)
        def _(r):
            @pl.loop(0, 128, step=16)
            def _(kk):
                pool_v[r, pl.ds(kk, 16)] = zv

        e_begin = rs_v[pl.ds(0, 16)][0]
        e_end = rs_v[pl.ds(nnodes, 16)][0]
        abegin = (e_begin // 16) * 16
        nch = (e_end - abegin + SEG_CH - 1) // SEG_CH

        def chunk_body(ch, st):
            base = abegin + ch * SEG_CH
            pltpu.sync_copy(c2_hbm.at[pl.ds(base, SEG_CH)], chunk_v)
            pltpu.sync_copy(ci_hbm.at[pl.ds(base, SEG_CH)],
                            c_v.at[pl.ds(0, SEG_CH)])

            lo = jnp.maximum(base, e_begin)
            hi = jnp.minimum(base + SEG_CH, e_end)

            def edge_body(e, st2):
                cprev, a0, a1, a2, a3 = st2
                row = e - base
                ce = c_v[pl.ds(row, 16)][0]
                same = ce == cprev
                v0 = chunk_v[row, pl.ds(0, 16)]
                v1 = chunk_v[row, pl.ds(16, 16)]
                v2 = chunk_v[row, pl.ds(32, 16)]
                v3 = chunk_v[row, pl.ds(48, 16)]
                a0 = jnp.where(same, jnp.maximum(a0, v0), v0)
                a1 = jnp.where(same, jnp.maximum(a1, v1), v1)
                a2 = jnp.where(same, jnp.maximum(a2, v2), v2)
                a3 = jnp.where(same, jnp.maximum(a3, v3), v3)
                lrow = ce - nlo
                pool_v[lrow, pl.ds(0, 16)] = a0
                pool_v[lrow, pl.ds(16, 16)] = a1
                pool_v[lrow, pl.ds(32, 16)] = a2
                pool_v[lrow, pl.ds(48, 16)] = a3
                return (ce, a0, a1, a2, a3)

            return jax.lax.fori_loop(lo, hi, edge_body, st)

        st0 = (jnp.int32(-1), zv, zv, zv, zv)
        jax.lax.fori_loop(0, nch, chunk_body, st0)
        pltpu.sync_copy(pool_v.at[pl.ds(0, NPW)], o_hbm.at[pl.ds(nlo, NPW)])

    return k(comb2, cIdxs, rs_pad)


def kernel(detFeatures, cIdxs, nIdxs, pairFeatRaw, pw_W0, pw_b0, pw_W1,
           pw_b1, pw_W2, pw_b2, blk_fc1_W, blk_fc1_b, blk_pw1_W, blk_pw1_b,
           blk_pw2_W, blk_pw2_b, blk_po1_W, blk_po1_b, blk_po2_W, blk_po2_b,
           blk_out_W, blk_out_b):
    E = cIdxs.shape[0]
    N = detFeatures.shape[0]
    p = _pairwise_mlp(pairFeatRaw, pw_W0, pw_b0, pw_W1, pw_b1, pw_W2, pw_b2)
    allIdx = jnp.concatenate([cIdxs, nIdxs]).reshape(1, 2 * E)
    rs = jnp.searchsorted(cIdxs, jnp.arange(N + 1, dtype=jnp.int32)
                          ).astype(jnp.int32)
    rs_pad = jnp.concatenate(
        [rs, jnp.full((32 * NPW + 8 - (N + 1),), E, jnp.int32)])
    ci_pad = jnp.concatenate([cIdxs, jnp.full((SEG_CH,), N, jnp.int32)])

    x = detFeatures
    pooled = None
    for i in range(NB):
        w1 = blk_fc1_W[i]
        b1 = blk_fc1_b[i].reshape(1, 32)
        wp = blk_pw1_W[i, :32, :]
        wc = blk_pw1_W[i, 32:64, :]
        wn = blk_pw1_W[i, 64:, :]
        pb1 = blk_pw1_b[i].reshape(1, 64)
        w2 = blk_pw2_W[i]
        pb2 = blk_pw2_b[i].reshape(1, 64)
        if i == 0:
            table = _node_table(x, w1, b1, wc, wn)
        else:
            x, table = _update_and_table(
                x, pooled, blk_po1_W[i - 1], blk_po1_b[i - 1].reshape(1, 64),
                blk_po2_W[i - 1], blk_po2_b[i - 1].reshape(1, 64),
                blk_out_W[i - 1], blk_out_b[i - 1].reshape(1, 128),
                w1, b1, wc, wn)
        z = _sc_gather(table, allIdx)
        comb2 = _edge_mlp(z, p, wp, pb1, w2, pb2, E)
        pooled = _sc_segmax(comb2, ci_pad, rs_pad, N, E)
    x = _final_update(
        x, pooled, blk_po1_W[NB - 1], blk_po1_b[NB - 1].reshape(1, 64),
        blk_po2_W[NB - 1], blk_po2_b[NB - 1].reshape(1, 64),
        blk_out_W[NB - 1], blk_out_b[NB - 1].reshape(1, 128),
    )
    return x
